# Initial kernel scaffold; baseline (speedup 1.0000x reference)
import jax
import jax.numpy as jnp
import numpy as np
from jax.experimental import pallas as pl
from jax.experimental.pallas import tpu as pltpu

IMAGE_W = 640
IMAGE_H = 480
FOV = 55.0
FOCAL = np.float32(IMAGE_W / (2.0 * np.tan(np.radians(FOV / 2.0))))
NG = 2048
TH = 16
NROWS = 16


def _project_kernel(px, py, pz, s0, s1, s2, c0, c1, c2, op, cp, out):
    t0, t1, t2 = cp[0, 3], cp[1, 3], cp[2, 3]
    dx = px[...] - t0
    dy = py[...] - t1
    dz = pz[...] - t2
    camx = dx * cp[0, 0] + dy * cp[0, 1] + dz * cp[0, 2]
    camy = dx * cp[1, 0] + dy * cp[1, 1] + dz * cp[1, 2]
    camz = dx * cp[2, 0] + dy * cp[2, 1] + dz * cp[2, 2]
    depth = jnp.maximum(-camz, 0.1)
    sx = FOCAL * camx / depth + IMAGE_W / 2.0
    sy = FOCAL * camy / depth + IMAGE_H / 2.0
    valid = (sx >= 0) & (sx < IMAGE_W) & (sy >= 0) & (sy < IMAGE_H)
    sm = (s0[...] + s1[...] + s2[...]) / 3.0
    ss = jnp.clip(sm * FOCAL / depth, 1.0, 20.0)
    radf = jnp.floor(ss * 3.0)
    xi = jnp.clip(jnp.floor(sx), 0.0, IMAGE_W - 1.0)
    yi = jnp.clip(jnp.floor(sy), 0.0, IMAGE_H - 1.0)
    big = jnp.float32(1e9)
    lox = jnp.where(valid, xi - radf, big)
    hix = jnp.where(valid, xi + radf, -big)
    loy = jnp.where(valid, yi - radf, big)
    hiy = jnp.where(valid, yi + radf, -big)
    opv = jnp.where(valid, op[...], 0.0)
    ninv = -0.5 / (ss * ss)

    def put(r, v):
        out[r] = v.reshape(1, NG)

    put(0, sx)
    put(1, sy)
    put(2, ninv)
    put(3, opv)
    put(4, lox)
    put(5, hix)
    put(6, loy)
    put(7, hiy)
    put(8, c0[...])
    put(9, c1[...])
    put(10, c2[...])
    put(11, jnp.zeros_like(sx))
    put(12, jnp.zeros_like(sx))
    put(13, jnp.zeros_like(sx))
    put(14, jnp.zeros_like(sx))
    put(15, jnp.zeros_like(sx))


def _raster_kernel(params, out):
    y0 = pl.program_id(0) * TH
    y0f = jnp.float32(0) + y0
    y1f = y0f + (TH - 1)
    pxf = jax.lax.broadcasted_iota(jnp.float32, (1, IMAGE_W), 1)
    pyf = y0f + jax.lax.broadcasted_iota(jnp.float32, (TH, 1), 0)
    out[...] = jnp.zeros((3, TH, IMAGE_W), jnp.float32)

    def body(k, _):
        loy = params[6, k]
        hiy = params[7, k]

        @pl.when((hiy >= y0f) & (loy <= y1f))
        def _():
            sx = params[0, k]
            sy = params[1, k]
            ninv = params[2, k]
            opv = params[3, k]
            lox = params[4, k]
            hix = params[5, k]
            ddx = pxf - sx
            wx = jnp.where((pxf >= lox) & (pxf <= hix),
                           jnp.exp(ninv * (ddx * ddx)), 0.0)
            ddy = pyf - sy
            wy = jnp.where((pyf >= loy) & (pyf <= hiy),
                           jnp.exp(ninv * (ddy * ddy)), 0.0)
            a = (opv * wy) * wx
            t = 1.0 - a
            out[0] = out[0] * t + a * params[8, k]
            out[1] = out[1] * t + a * params[9, k]
            out[2] = out[2] * t + a * params[10, k]

        return 0

    jax.lax.fori_loop(0, NG, body, 0)


def kernel(positions, scales, rotations, colors, opacities, camera_pose):
    del rotations
    plane = lambda a: a.reshape(8, 256)
    args = [plane(positions[:, 0]), plane(positions[:, 1]), plane(positions[:, 2]),
            plane(scales[:, 0]), plane(scales[:, 1]), plane(scales[:, 2]),
            plane(colors[:, 0]), plane(colors[:, 1]), plane(colors[:, 2]),
            plane(opacities), camera_pose]
    vspec = pl.BlockSpec((8, 256), lambda: (0, 0))
    params = pl.pallas_call(
        _project_kernel,
        out_shape=jax.ShapeDtypeStruct((NROWS, NG), jnp.float32),
        in_specs=[vspec] * 10 + [pl.BlockSpec(memory_space=pltpu.SMEM)],
        out_specs=pl.BlockSpec((NROWS, NG), lambda: (0, 0)),
    )(*args)

    img = pl.pallas_call(
        _raster_kernel,
        grid=(IMAGE_H // TH,),
        out_shape=jax.ShapeDtypeStruct((3, IMAGE_H, IMAGE_W), jnp.float32),
        in_specs=[pl.BlockSpec((NROWS, NG), lambda i: (0, 0),
                               memory_space=pltpu.SMEM)],
        out_specs=pl.BlockSpec((3, TH, IMAGE_W), lambda i: (0, i, 0)),
    )(img := None) if False else pl.pallas_call(
        _raster_kernel,
        grid=(IMAGE_H // TH,),
        out_shape=jax.ShapeDtypeStruct((3, IMAGE_H, IMAGE_W), jnp.float32),
        in_specs=[pl.BlockSpec((NROWS, NG), lambda i: (0, 0),
                               memory_space=pltpu.SMEM)],
        out_specs=pl.BlockSpec((3, TH, IMAGE_W), lambda i: (0, i, 0)),
    )(params)
    return img


# speedup vs baseline: 38.7880x; 38.7880x over previous
import jax
import jax.numpy as jnp
import numpy as np
from jax.experimental import pallas as pl
from jax.experimental.pallas import tpu as pltpu

IMAGE_W = 640
IMAGE_H = 480
FOV = 55.0
FOCAL = np.float32(IMAGE_W / (2.0 * np.tan(np.radians(FOV / 2.0))))
NG = 2048
TH = 16
NROWS = 16


def _project_kernel(camx_r, camy_r, camz_r, s0, s1, s2, c0, c1, c2, op, cp, out):
    camx = camx_r[...]
    camy = camy_r[...]
    camz = camz_r[...]
    depth = jnp.maximum(-camz, 0.1)
    sx = FOCAL * camx / depth + IMAGE_W / 2.0
    sy = FOCAL * camy / depth + IMAGE_H / 2.0
    valid = (sx >= 0) & (sx < IMAGE_W) & (sy >= 0) & (sy < IMAGE_H)
    sm = (s0[...] + s1[...] + s2[...]) / 3.0
    ss = jnp.clip(sm * FOCAL / depth, 1.0, 20.0)
    radf = jnp.floor(ss * 3.0)
    xi = jnp.clip(jnp.floor(sx), 0.0, IMAGE_W - 1.0)
    yi = jnp.clip(jnp.floor(sy), 0.0, IMAGE_H - 1.0)
    big = jnp.float32(1e9)
    lox = jnp.where(valid, xi - radf, big)
    hix = jnp.where(valid, xi + radf, -big)
    loy = jnp.where(valid, yi - radf, big)
    hiy = jnp.where(valid, yi + radf, -big)
    opv = jnp.where(valid, op[...], 0.0)
    ninv = -0.5 / (ss * ss)

    def put(r, v):
        out[pl.ds(r, 1), :] = v.reshape(1, NG)

    put(0, sx)
    put(1, sy)
    put(2, ninv)
    put(3, opv)
    put(4, lox)
    put(5, hix)
    put(6, loy)
    put(7, hiy)
    put(8, c0[...])
    put(9, c1[...])
    put(10, c2[...])
    put(11, jnp.zeros_like(sx))
    put(12, jnp.zeros_like(sx))
    put(13, jnp.zeros_like(sx))
    put(14, jnp.zeros_like(sx))
    put(15, jnp.zeros_like(sx))


def _raster_kernel(params, out):
    y0 = pl.program_id(0) * TH
    y0f = jnp.float32(0) + y0
    y1f = y0f + (TH - 1)
    pxf = jax.lax.broadcasted_iota(jnp.int32, (1, IMAGE_W), 1).astype(jnp.float32)
    pyf = y0f + jax.lax.broadcasted_iota(jnp.int32, (TH, 1), 0).astype(jnp.float32)
    out[...] = jnp.zeros((3, TH, IMAGE_W), jnp.float32)

    def body(k, _):
        loy = params[6, k]
        hiy = params[7, k]

        @pl.when((hiy >= y0f) & (loy <= y1f))
        def _():
            sx = params[0, k]
            sy = params[1, k]
            ninv = params[2, k]
            opv = params[3, k]
            lox = params[4, k]
            hix = params[5, k]
            ddx = pxf - sx
            wx = jnp.where((pxf >= lox) & (pxf <= hix),
                           jnp.exp(ninv * (ddx * ddx)), 0.0)
            ddy = pyf - sy
            wy = jnp.where((pyf >= loy) & (pyf <= hiy),
                           jnp.exp(ninv * (ddy * ddy)), 0.0)
            a = (opv * wy) * wx
            t = 1.0 - a
            out[0] = out[0] * t + a * params[8, k]
            out[1] = out[1] * t + a * params[9, k]
            out[2] = out[2] * t + a * params[10, k]

        return 0

    jax.lax.fori_loop(0, NG, body, 0)


def kernel(positions, scales, rotations, colors, opacities, camera_pose):
    del rotations
    R = camera_pose[:3, :3]
    t = camera_pose[:3, 3]
    cam = (positions - t) @ R.T
    plane = lambda a: a.reshape(8, 256)
    args = [plane(cam[:, 0]), plane(cam[:, 1]), plane(cam[:, 2]),
            plane(scales[:, 0]), plane(scales[:, 1]), plane(scales[:, 2]),
            plane(colors[:, 0]), plane(colors[:, 1]), plane(colors[:, 2]),
            plane(opacities), camera_pose]
    vspec = pl.BlockSpec((8, 256), lambda: (0, 0))
    params = pl.pallas_call(
        _project_kernel,
        out_shape=jax.ShapeDtypeStruct((NROWS, NG), jnp.float32),
        in_specs=[vspec] * 10 + [pl.BlockSpec(memory_space=pltpu.SMEM)],
        out_specs=pl.BlockSpec((NROWS, NG), lambda: (0, 0)),
    )(*args)

    img = pl.pallas_call(
        _raster_kernel,
        grid=(IMAGE_H // TH,),
        out_shape=jax.ShapeDtypeStruct((3, IMAGE_H, IMAGE_W), jnp.float32),
        in_specs=[pl.BlockSpec((NROWS, NG), lambda i: (0, 0),
                               memory_space=pltpu.SMEM)],
        out_specs=pl.BlockSpec((3, TH, IMAGE_W), lambda i: (0, i, 0)),
    )(params)
    return img


# speedup vs baseline: 429.8975x; 11.0833x over previous
import jax
import jax.numpy as jnp
import numpy as np
from jax.experimental import pallas as pl
from jax.experimental.pallas import tpu as pltpu

IMAGE_W = 640
IMAGE_H = 480
FOV = 55.0
FOCAL = np.float32(IMAGE_W / (2.0 * np.tan(np.radians(FOV / 2.0))))
NG = 2048
TH = 16
NROWS = 16


def _project_kernel(camx_r, camy_r, camz_r, s0, s1, s2, c0, c1, c2, op, cp, out):
    camx = camx_r[...]
    camy = camy_r[...]
    camz = camz_r[...]
    depth = jnp.maximum(-camz, 0.1)
    sx = FOCAL * camx / depth + IMAGE_W / 2.0
    sy = FOCAL * camy / depth + IMAGE_H / 2.0
    valid = (sx >= 0) & (sx < IMAGE_W) & (sy >= 0) & (sy < IMAGE_H)
    sm = (s0[...] + s1[...] + s2[...]) / 3.0
    ss = jnp.clip(sm * FOCAL / depth, 1.0, 20.0)
    radf = jnp.floor(ss * 3.0)
    xi = jnp.clip(jnp.floor(sx), 0.0, IMAGE_W - 1.0)
    yi = jnp.clip(jnp.floor(sy), 0.0, IMAGE_H - 1.0)
    big = jnp.float32(1e9)
    lox = jnp.where(valid, xi - radf, big)
    hix = jnp.where(valid, xi + radf, -big)
    loy = jnp.where(valid, yi - radf, big)
    hiy = jnp.where(valid, yi + radf, -big)
    opv = jnp.where(valid, op[...], 0.0)
    ninv = -0.5 / (ss * ss)

    def put(r, v):
        out[pl.ds(r, 1), :] = v.reshape(1, NG)

    put(0, sx)
    put(1, sy)
    put(2, ninv)
    put(3, opv)
    put(4, lox)
    put(5, hix)
    put(6, loy)
    put(7, hiy)
    put(8, c0[...])
    put(9, c1[...])
    put(10, c2[...])
    put(11, jnp.zeros_like(sx))
    put(12, jnp.zeros_like(sx))
    put(13, jnp.zeros_like(sx))
    put(14, jnp.zeros_like(sx))
    put(15, jnp.zeros_like(sx))


def _compact_kernel(params, cparams, cnt):
    def body(k, c):
        pred = params[7, k] >= params[6, k]

        @pl.when(pred)
        def _():
            for r in range(11):
                cparams[r, c] = params[r, k]

        return c + jnp.where(pred, 1, 0)

    n = jax.lax.fori_loop(0, NG, body, jnp.int32(0))
    cnt[0, 0] = n


def _raster_kernel(cparams, cnt, out, tbuf):
    y0 = pl.program_id(0) * TH
    y0f = jnp.float32(0) + y0
    y1f = y0f + (TH - 1)
    pxf = jax.lax.broadcasted_iota(jnp.int32, (1, IMAGE_W), 1).astype(jnp.float32)
    pyf = y0f + jax.lax.broadcasted_iota(jnp.int32, (TH, 1), 0).astype(jnp.float32)
    out[...] = jnp.zeros((3, TH, IMAGE_W), jnp.float32)
    tbuf[...] = jnp.ones((TH, IMAGE_W), jnp.float32)
    n = cnt[0, 0]

    def body(i, _):
        k = n - 1 - i
        loy = cparams[6, k]
        hiy = cparams[7, k]

        @pl.when((hiy >= y0f) & (loy <= y1f))
        def _():
            sx = cparams[0, k]
            sy = cparams[1, k]
            ninv = cparams[2, k]
            opv = cparams[3, k]
            lox = cparams[4, k]
            hix = cparams[5, k]
            ddx = pxf - sx
            wx = jnp.where((pxf >= lox) & (pxf <= hix),
                           opv * jnp.exp(ninv * (ddx * ddx)), 0.0)
            ddy = pyf - sy
            wy = jnp.where((pyf >= loy) & (pyf <= hiy),
                           jnp.exp(ninv * (ddy * ddy)), 0.0)
            am = (wy * wx) * tbuf[...]
            tbuf[...] = tbuf[...] - am
            out[0] = out[0] + am * cparams[8, k]
            out[1] = out[1] + am * cparams[9, k]
            out[2] = out[2] + am * cparams[10, k]

        return 0

    jax.lax.fori_loop(0, n, body, 0)


def kernel(positions, scales, rotations, colors, opacities, camera_pose):
    del rotations
    R = camera_pose[:3, :3]
    t = camera_pose[:3, 3]
    cam = (positions - t) @ R.T
    plane = lambda a: a.reshape(8, 256)
    args = [plane(cam[:, 0]), plane(cam[:, 1]), plane(cam[:, 2]),
            plane(scales[:, 0]), plane(scales[:, 1]), plane(scales[:, 2]),
            plane(colors[:, 0]), plane(colors[:, 1]), plane(colors[:, 2]),
            plane(opacities), camera_pose]
    vspec = pl.BlockSpec((8, 256), lambda: (0, 0))
    params = pl.pallas_call(
        _project_kernel,
        out_shape=jax.ShapeDtypeStruct((NROWS, NG), jnp.float32),
        in_specs=[vspec] * 10 + [pl.BlockSpec(memory_space=pltpu.SMEM)],
        out_specs=pl.BlockSpec((NROWS, NG), lambda: (0, 0)),
    )(*args)

    smem_spec = pl.BlockSpec(memory_space=pltpu.SMEM)
    cparams, cnt = pl.pallas_call(
        _compact_kernel,
        out_shape=(jax.ShapeDtypeStruct((NROWS, NG), jnp.float32),
                   jax.ShapeDtypeStruct((1, 1), jnp.int32)),
        in_specs=[smem_spec],
        out_specs=(smem_spec, smem_spec),
    )(params)

    img = pl.pallas_call(
        _raster_kernel,
        grid=(IMAGE_H // TH,),
        out_shape=jax.ShapeDtypeStruct((3, IMAGE_H, IMAGE_W), jnp.float32),
        in_specs=[pl.BlockSpec((NROWS, NG), lambda i: (0, 0),
                               memory_space=pltpu.SMEM),
                  pl.BlockSpec((1, 1), lambda i: (0, 0),
                               memory_space=pltpu.SMEM)],
        out_specs=pl.BlockSpec((3, TH, IMAGE_W), lambda i: (0, i, 0)),
        scratch_shapes=[pltpu.VMEM((TH, IMAGE_W), jnp.float32)],
    )(cparams, cnt)
    return img
